# baseline (speedup 1.0000x reference)
import functools

import jax
import jax.numpy as jnp
import numpy as np
from jax import lax
from jax.experimental import pallas as pl
from jax.experimental.pallas import tpu as pltpu
from jax.experimental.pallas import tpu_sc as plsc

F32 = jnp.float32
I32 = jnp.int32

N = 10000
NP = 10240
NR = 10016
NRT = NR // 16
E = 330000
NTILE = 16
NCORE = 2
NW = NTILE * NCORE
EPT = 20736
NCH = EPT // 128
EP = EPT * NTILE
EPW = EP // NW
DEN_R = NP // 16


def _gat_dense_first(x, w, a_src, a_dst, dh):
    nb = NP // 512
    di = x.shape[1]
    do = 2 * dh

    kq = dh // 32

    def body(x_ref, w_ref, asr_ref, adr_ref, hs_ref, as_ref, ad_ref):
        h = jnp.dot(x_ref[...], w_ref[...], preferred_element_type=F32)
        for ci in range(2):
            for q in range(kq):
                lo = ci * dh + q * 32
                hs_ref[ci, q] = h[:, lo:lo + 32]
        as_ref[...] = jnp.dot(h, asr_ref[...], preferred_element_type=F32)
        ad_ref[...] = jnp.dot(h, adr_ref[...], preferred_element_type=F32)

    return pl.pallas_call(
        body,
        grid=(nb,),
        in_specs=[
            pl.BlockSpec((512, di), lambda i: (i, 0)),
            pl.BlockSpec((di, do), lambda i: (0, 0)),
            pl.BlockSpec((do, 1), lambda i: (0, 0)),
            pl.BlockSpec((do, 1), lambda i: (0, 0)),
        ],
        out_specs=[
            pl.BlockSpec((2, kq, 512, 32), lambda i: (0, 0, i, 0)),
            pl.BlockSpec((512, 1), lambda i: (i, 0)),
            pl.BlockSpec((512, 1), lambda i: (i, 0)),
        ],
        out_shape=[
            jax.ShapeDtypeStruct((2, kq, NP, 32), F32),
            jax.ShapeDtypeStruct((NP, 1), F32),
            jax.ShapeDtypeStruct((NP, 1), F32),
        ],
    )(x, w, a_src, a_dst)


def _gat_dense_mid(s_prev, b_prev, w, a_src, a_dst, dh):
    nb = NP // 512
    kp = s_prev.shape[1]
    dp = kp * 32
    di = 2 * dp
    kq = dh // 32

    def body(s_ref, b_ref, w_ref, asr_ref, adr_ref, hs_ref, as_ref, ad_ref):
        x0 = jnp.concatenate(
            [jnp.maximum(s_ref[0, q] + b_ref[0, q], 0.0) for q in range(kp)],
            axis=1)
        x1 = jnp.concatenate(
            [jnp.maximum(s_ref[1, q] + b_ref[1, q], 0.0) for q in range(kp)],
            axis=1)
        h = (jnp.dot(x0, w_ref[0:dp, :], preferred_element_type=F32)
             + jnp.dot(x1, w_ref[dp:di, :], preferred_element_type=F32))
        for ci in range(2):
            for q in range(kq):
                lo = ci * dh + q * 32
                hs_ref[ci, q] = h[:, lo:lo + 32]
        as_ref[...] = jnp.dot(h, asr_ref[...], preferred_element_type=F32)
        ad_ref[...] = jnp.dot(h, adr_ref[...], preferred_element_type=F32)

    return pl.pallas_call(
        body,
        grid=(nb,),
        in_specs=[
            pl.BlockSpec((2, kp, 512, 32), lambda i: (0, 0, i, 0)),
            pl.BlockSpec((2, kp, 1, 32), lambda i: (0, 0, 0, 0)),
            pl.BlockSpec((di, 2 * dh), lambda i: (0, 0)),
            pl.BlockSpec((2 * dh, 1), lambda i: (0, 0)),
            pl.BlockSpec((2 * dh, 1), lambda i: (0, 0)),
        ],
        out_specs=[
            pl.BlockSpec((2, kq, 512, 32), lambda i: (0, 0, i, 0)),
            pl.BlockSpec((512, 1), lambda i: (i, 0)),
            pl.BlockSpec((512, 1), lambda i: (i, 0)),
        ],
        out_shape=[
            jax.ShapeDtypeStruct((2, kq, NP, 32), F32),
            jax.ShapeDtypeStruct((NP, 1), F32),
            jax.ShapeDtypeStruct((NP, 1), F32),
        ],
    )(s_prev, b_prev, w, a_src, a_dst)


def _sc_pass_a(src_f, dst_f, as_f, ad_f, rid):
    mesh = plsc.VectorSubcoreMesh(core_axis_name="c", subcore_axis_name="s")

    @functools.partial(
        pl.kernel,
        mesh=mesh,
        compiler_params=pltpu.CompilerParams(needs_layout_passes=False, use_tc_tiling_on_sc=False),
        out_type=[
            jax.ShapeDtypeStruct((EP,), F32),
            jax.ShapeDtypeStruct((2, DEN_R, 16), F32),
        ],
        scratch_types=[
            pltpu.VMEM((EPW,), I32),
            pltpu.VMEM((EPW,), I32),
            pltpu.VMEM((EPW,), F32),
            pltpu.VMEM((NP,), F32),
            pltpu.VMEM((NP,), F32),
            pltpu.VMEM((DEN_R, 16), F32),
            pltpu.VMEM((5, 128), I32),
            pltpu.VMEM_SHARED((DEN_R, 16), F32),
        ],
    )
    def k(src_h, dst_h, as_h, ad_h, rid_h, ee_h, den_h,
          srcv, dstv, eev, asv, adv, denl, ridv, den_sh):
        c = lax.axis_index("c")
        s = lax.axis_index("s")
        w = s * NCORE + c
        off = w * EPW
        pltpu.sync_copy(as_h, asv)
        pltpu.sync_copy(ad_h, adv)
        pltpu.sync_copy(src_h.at[pl.ds(off, EPW)], srcv)
        pltpu.sync_copy(dst_h.at[pl.ds(off, EPW)], dstv)
        pltpu.sync_copy(rid_h, ridv)

        zero16 = jnp.zeros((16,), F32)

        def zloop(i, carry):
            denl[i] = zero16
            return carry

        lax.fori_loop(0, DEN_R, zloop, 0)

        def eloop(v, carry):
            sv = srcv[pl.ds(v * 16, 16)]
            dv = dstv[pl.ds(v * 16, 16)]
            a1 = plsc.load_gather(asv, [sv])
            a2 = plsc.load_gather(adv, [dv])
            e = a1 + a2
            e = jnp.maximum(e, e * 0.2)
            ee = jnp.exp(e)
            eev[pl.ds(v * 16, 16)] = ee
            plsc.addupdate_scatter(
                denl, [jnp.right_shift(dv, 4), jnp.bitwise_and(dv, 15)], ee)
            return carry

        lax.fori_loop(0, EPW // 16, eloop, 0)

        pltpu.sync_copy(eev, ee_h.at[pl.ds(off, EPW)])

        @pl.when(s == 0)
        def _():
            pltpu.sync_copy(denl, den_sh)

        plsc.subcore_barrier()

        @pl.when(s != 0)
        def _():
            for j in range(5):
                pltpu.sync_copy(denl.at[pl.ds(j * 128, 128)],
                                den_sh.at[ridv.at[j]], add=True)

        plsc.subcore_barrier()
        rows = DEN_R // NTILE
        pltpu.sync_copy(den_sh.at[pl.ds(s * rows, rows)],
                        den_h.at[c, pl.ds(s * rows, rows)])

    return k(src_f, dst_f, as_f, ad_f, rid)


def _sc_pass_b(src3, dst3, ee_f, den2, h_split, dh):
    mesh = plsc.VectorSubcoreMesh(core_axis_name="c", subcore_axis_name="s")

    kq = dh // 32

    @functools.partial(
        pl.kernel,
        mesh=mesh,
        compiler_params=pltpu.CompilerParams(needs_layout_passes=False,
                                             use_tc_tiling_on_sc=False),
        out_type=[
            jax.ShapeDtypeStruct((EP,), F32),
            jax.ShapeDtypeStruct((2, kq, NP, 32), F32),
        ],
        scratch_types=[
            pltpu.VMEM((NCH, 128), I32),
            pltpu.VMEM((NCH, 128), I32),
            pltpu.VMEM((EPT,), F32),
            pltpu.VMEM((DEN_R, 16), F32),
            pltpu.VMEM((DEN_R, 16), F32),
            pltpu.VMEM((128, 32), F32),
            pltpu.VMEM_SHARED((NR, 32), F32),
            pltpu.SemaphoreType.DMA,
        ],
    )
    def k(src_h, dst_h, ee_h, den_h, hs_h, alpha_h, s_h,
          srcv, dstv, av, d0v, d1v, rowb, out_sh, sem):
        c = lax.axis_index("c")
        s = lax.axis_index("s")
        off = s * EPT
        pltpu.sync_copy(src_h.at[s], srcv)
        pltpu.sync_copy(dst_h.at[s], dstv)
        pltpu.sync_copy(ee_h.at[pl.ds(off, EPT)], av)
        pltpu.sync_copy(den_h.at[0], d0v)
        pltpu.sync_copy(den_h.at[1], d1v)

        def aloop(j, carry):
            for t in range(8):
                dv = dstv[j, pl.ds(t * 16, 16)]
                hi = jnp.right_shift(dv, 4)
                lo = jnp.bitwise_and(dv, 15)
                d0 = plsc.load_gather(d0v, [hi, lo])
                d1 = plsc.load_gather(d1v, [hi, lo])
                ee = av[pl.ds(j * 128 + t * 16, 16)]
                av[pl.ds(j * 128 + t * 16, 16)] = ee / (d0 + d1)
            return carry

        lax.fori_loop(0, NCH, aloop, 0)

        @pl.when(c == 0)
        def _():
            pltpu.sync_copy(av, alpha_h.at[pl.ds(off, EPT)])

        zero16 = jnp.zeros((16,), F32)
        base = s * NRT

        for q in range(kq):
            def zloop(i, carry):
                rowb[i, pl.ds(0, 16)] = zero16
                rowb[i, pl.ds(16, 16)] = zero16
                return carry

            lax.fori_loop(0, 128, zloop, 0)
            for z in range(4):
                pltpu.sync_copy(rowb, out_sh.at[pl.ds(base + z * 128, 128)])
            pltpu.sync_copy(rowb.at[pl.ds(0, NRT - 512)],
                            out_sh.at[pl.ds(base + 512, NRT - 512)])
            plsc.subcore_barrier()

            def chunk(j, carry):
                pltpu.async_copy(hs_h.at[c, q].at[srcv.at[j]], rowb,
                                 sem).wait()

                def scale(rb, carry2):
                    a16 = av[pl.ds(j * 128 + rb * 16, 16)]
                    for u in range(16):
                        a = a16[u]
                        r = rb * 16 + u
                        rowb[r, pl.ds(0, 16)] = rowb[r, pl.ds(0, 16)] * a
                        rowb[r, pl.ds(16, 16)] = rowb[r, pl.ds(16, 16)] * a
                    return carry2

                lax.fori_loop(0, 8, scale, 0)
                pltpu.sync_copy(rowb, out_sh.at[dstv.at[j]], add=True)
                return carry

            lax.fori_loop(0, NCH, chunk, 0)

            plsc.subcore_barrier()
            pltpu.sync_copy(out_sh.at[pl.ds(base, NRT)],
                            s_h.at[c, q, pl.ds(base, NRT)])

    return k(src3, dst3, ee_f, den2, h_split)


def _lstm_u_first(s3, b3, wt, bias):
    nb = NP // 512
    kp = s3.shape[1]
    dp = kp * 32

    def body(s_ref, b_ref, w_ref, bias_ref, u_ref):
        x0 = jnp.concatenate(
            [jnp.maximum(s_ref[0, q] + b_ref[0, q], 0.0) for q in range(kp)],
            axis=1)
        x1 = jnp.concatenate(
            [jnp.maximum(s_ref[1, q] + b_ref[1, q], 0.0) for q in range(kp)],
            axis=1)
        u_ref[...] = (jnp.dot(x0, w_ref[0:dp, :], preferred_element_type=F32)
                      + jnp.dot(x1, w_ref[dp:2 * dp, :],
                                preferred_element_type=F32)
                      + bias_ref[...])

    return pl.pallas_call(
        body,
        grid=(nb,),
        in_specs=[
            pl.BlockSpec((2, kp, 512, 32), lambda i: (0, 0, i, 0)),
            pl.BlockSpec((2, kp, 1, 32), lambda i: (0, 0, 0, 0)),
            pl.BlockSpec((2 * dp, 1024), lambda i: (0, 0)),
            pl.BlockSpec((1, 1024), lambda i: (0, 0)),
        ],
        out_specs=pl.BlockSpec((512, 1024), lambda i: (i, 0)),
        out_shape=jax.ShapeDtypeStruct((NP, 1024), F32),
    )(s3, b3, wt, bias)


def _lstm_u_mid(yf, yb, wt, bias):
    nb = NP // 512

    def body(yf_ref, yb_ref, w_ref, bias_ref, u_ref):
        u_ref[...] = (jnp.dot(yf_ref[...], w_ref[0:128, :],
                              preferred_element_type=F32)
                      + jnp.dot(yb_ref[...], w_ref[128:256, :],
                                preferred_element_type=F32)
                      + bias_ref[...])

    return pl.pallas_call(
        body,
        grid=(nb,),
        in_specs=[
            pl.BlockSpec((512, 128), lambda i: (i, 0)),
            pl.BlockSpec((512, 128), lambda i: (i, 0)),
            pl.BlockSpec((256, 1024), lambda i: (0, 0)),
            pl.BlockSpec((1, 1024), lambda i: (0, 0)),
        ],
        out_specs=pl.BlockSpec((512, 1024), lambda i: (i, 0)),
        out_shape=jax.ShapeDtypeStruct((NP, 1024), F32),
    )(yf, yb, wt, bias)


def _lstm_rec(u, whf, whb):
    tb = 1000
    nblk = N // tb

    def body(uf_ref, ub_ref, wf_ref, wb_ref, yf_ref, yb_ref, st_s):
        i = pl.program_id(0)

        @pl.when(i == 0)
        def _():
            st_s[...] = jnp.zeros((8, 128), F32)

        def step(r, carry):
            h2, c2 = carry
            hbf = h2.astype(jnp.bfloat16)
            gf = (jnp.dot(hbf[0:1], wf_ref[...], preferred_element_type=F32)
                  + uf_ref[pl.ds(r, 1), :])
            gb = (jnp.dot(hbf[1:2], wb_ref[...], preferred_element_type=F32)
                  + ub_ref[pl.ds(tb - 1 - r, 1), :])
            g2 = jnp.concatenate([gf, gb], axis=0)
            s2 = jax.nn.sigmoid(g2[:, 0:384])
            t2 = jnp.tanh(g2[:, 384:512])
            c2 = s2[:, 128:256] * c2 + s2[:, 0:128] * t2
            h2 = s2[:, 256:384] * jnp.tanh(c2)
            yf_ref[pl.ds(r, 1), :] = h2[0:1]
            yb_ref[pl.ds(tb - 1 - r, 1), :] = h2[1:2]
            return h2, c2

        init = (st_s[0:2, :], st_s[2:4, :])
        h2, c2 = lax.fori_loop(0, tb, step, init, unroll=4)
        st_s[0:2, :] = h2
        st_s[2:4, :] = c2

    return pl.pallas_call(
        body,
        grid=(nblk,),
        in_specs=[
            pl.BlockSpec((tb, 512), lambda i: (i, 0)),
            pl.BlockSpec((tb, 512), lambda i: (nblk - 1 - i, 1)),
            pl.BlockSpec((128, 512), lambda i: (0, 0)),
            pl.BlockSpec((128, 512), lambda i: (0, 0)),
        ],
        out_specs=[
            pl.BlockSpec((tb, 128), lambda i: (i, 0)),
            pl.BlockSpec((tb, 128), lambda i: (nblk - 1 - i, 0)),
        ],
        out_shape=[
            jax.ShapeDtypeStruct((NP, 128), F32),
            jax.ShapeDtypeStruct((NP, 128), F32),
        ],
        scratch_shapes=[
            pltpu.VMEM((8, 128), F32),
        ],
    )(u, u, whf, whb)


def _pool_fc(yf, yb, bmat, fc1w, fc1b, g1, b1, fc2w, fc2b, g2, b2,
             fc3w, fc3b):
    rb = 400
    nb = N // rb
    bn_scale = float(1.0 / np.sqrt(1.0 + 1e-5))

    def body(yf_ref, yb_ref, b_ref, fc1w_ref, fc1b_ref, g1_ref, b1_ref,
             fc2w_ref, fc2b_ref, g2_ref, b2_ref, fc3w_ref, fc3b_ref,
             z_ref, acc):
        i = pl.program_id(0)

        @pl.when(i == 0)
        def _():
            acc[...] = jnp.zeros((32, 256), F32)

        dn = (((0,), (0,)), ((), ()))
        acc[:, 0:128] += lax.dot_general(b_ref[...], yf_ref[...], dn,
                                         preferred_element_type=F32)
        acc[:, 128:256] += lax.dot_general(b_ref[...], yb_ref[...], dn,
                                           preferred_element_type=F32)

        @pl.when(i == nb - 1)
        def _():
            p = acc[...] * (1.0 / 500.0)
            z = jnp.dot(p, fc1w_ref[...], preferred_element_type=F32)
            z = jnp.maximum(z + fc1b_ref[...], 0.0)
            z = z * bn_scale * g1_ref[...] + b1_ref[...]
            z = jnp.dot(z, fc2w_ref[...], preferred_element_type=F32)
            z = jnp.maximum(z + fc2b_ref[...], 0.0)
            z = z * bn_scale * g2_ref[...] + b2_ref[...]
            z = jnp.dot(z, fc3w_ref[...], preferred_element_type=F32)
            z_ref[...] = z[0:20, :] + fc3b_ref[...]

    return pl.pallas_call(
        body,
        grid=(nb,),
        in_specs=[
            pl.BlockSpec((rb, 128), lambda i: (i, 0)),
            pl.BlockSpec((rb, 128), lambda i: (i, 0)),
            pl.BlockSpec((rb, 32), lambda i: (i, 0)),
            pl.BlockSpec((256, 256), lambda i: (0, 0)),
            pl.BlockSpec((1, 256), lambda i: (0, 0)),
            pl.BlockSpec((1, 256), lambda i: (0, 0)),
            pl.BlockSpec((1, 256), lambda i: (0, 0)),
            pl.BlockSpec((256, 64), lambda i: (0, 0)),
            pl.BlockSpec((1, 64), lambda i: (0, 0)),
            pl.BlockSpec((1, 64), lambda i: (0, 0)),
            pl.BlockSpec((1, 64), lambda i: (0, 0)),
            pl.BlockSpec((64, 64), lambda i: (0, 0)),
            pl.BlockSpec((1, 64), lambda i: (0, 0)),
        ],
        out_specs=pl.BlockSpec((20, 64), lambda i: (0, 0)),
        out_shape=jax.ShapeDtypeStruct((20, 64), F32),
        scratch_shapes=[pltpu.VMEM((32, 256), F32)],
    )(yf, yb, bmat, fc1w, fc1b, g1, b1, fc2w, fc2b, g2, b2, fc3w, fc3b)


def _gat_layer(x_or_s, b_prev, params, idx, layer, dh, first):
    w = params['W%d' % layer]
    a_src = params['a_src%d' % layer].reshape(-1, 1)
    a_dst = params['a_dst%d' % layer].reshape(-1, 1)
    if first:
        hs, as_, ad_ = _gat_dense_first(x_or_s, w, a_src, a_dst, dh)
    else:
        hs, as_, ad_ = _gat_dense_mid(x_or_s, b_prev, w, a_src, a_dst, dh)
    src_f, dst_f, src3, dst3, rid = idx
    ee, den2 = _sc_pass_a(src_f, dst_f, as_.reshape(NP), ad_.reshape(NP),
                          rid)
    alpha, s_out = _sc_pass_b(src3, dst3, ee, den2, hs, dh)
    return alpha, s_out


def kernel(x, edge_index, params):
    loop = jnp.arange(N, dtype=edge_index.dtype)
    ei = jnp.concatenate([edge_index, jnp.stack([loop, loop])], axis=1)

    src_f = jnp.concatenate([ei[0], jnp.zeros((EP - E,), I32)])
    dst_f = jnp.concatenate([ei[1], jnp.full((EP - E,), N, I32)])
    src3 = src_f.reshape(NTILE, NCH, 128)
    dst3 = dst_f.reshape(NTILE, NCH, 128)
    rid = jnp.arange(DEN_R, dtype=I32).reshape(5, 128)
    idx = (src_f, dst_f, src3, dst3, rid)

    x_p = jnp.pad(x, ((0, NP - N), (0, 0)))

    a1, s1 = _gat_layer(x_p, None, params, idx, 1, 64, True)
    b1 = params['b1'].reshape(2, 2, 1, 32)
    a2, s2 = _gat_layer(s1, b1, params, idx, 2, 128, False)
    b2 = params['b2'].reshape(2, 4, 1, 32)
    a3, s3 = _gat_layer(s2, b2, params, idx, 3, 64, False)
    b3 = params['b3'].reshape(2, 2, 1, 32)

    def _gp(m):
        return jnp.concatenate([m[0:256], m[384:512], m[256:384]], axis=0)

    w0 = jnp.concatenate([_gp(params['Wih_l0f']),
                          _gp(params['Wih_l0b'])], axis=0).T
    bias0 = (jnp.concatenate([_gp(params['bih_l0f'] + params['bhh_l0f']),
                              _gp(params['bih_l0b'] + params['bhh_l0b'])])
             .reshape(1, 1024))
    u0 = _lstm_u_first(s3, b3, w0, bias0)
    bf16 = jnp.bfloat16
    y0f, y0b = _lstm_rec(u0, _gp(params['Whh_l0f']).T.astype(bf16),
                         _gp(params['Whh_l0b']).T.astype(bf16))

    w1 = jnp.concatenate([_gp(params['Wih_l1f']),
                          _gp(params['Wih_l1b'])], axis=0).T
    bias1 = (jnp.concatenate([_gp(params['bih_l1f'] + params['bhh_l1f']),
                              _gp(params['bih_l1b'] + params['bhh_l1b'])])
             .reshape(1, 1024))
    u1 = _lstm_u_mid(y0f, y0b, w1, bias1)
    y1f, y1b = _lstm_rec(u1, _gp(params['Whh_l1f']).T.astype(bf16),
                         _gp(params['Whh_l1b']).T.astype(bf16))

    row = jnp.arange(NP, dtype=I32)
    grp = row // 500
    bmat = jnp.where((row[:, None] < N) & (grp[:, None] == jnp.arange(32)),
                     1.0, 0.0).astype(F32)
    z = _pool_fc(y1f, y1b, bmat,
                 params['fc1_w'], params['fc1_b'].reshape(1, 256),
                 params['bn1_g'].reshape(1, 256),
                 params['bn1_b'].reshape(1, 256),
                 params['fc2_w'], params['fc2_b'].reshape(1, 64),
                 params['bn2_g'].reshape(1, 64),
                 params['bn2_b'].reshape(1, 64),
                 params['fc3_w'], params['fc3_b'].reshape(1, 64))

    return (z, a1[:E], a2[:E], a3[:E], ei)


# speedup vs baseline: 1.0583x; 1.0583x over previous
import functools

import jax
import jax.numpy as jnp
import numpy as np
from jax import lax
from jax.experimental import pallas as pl
from jax.experimental.pallas import tpu as pltpu
from jax.experimental.pallas import tpu_sc as plsc

F32 = jnp.float32
I32 = jnp.int32

N = 10000
NP = 10240
NR = 10016
NRT = NR // 16
E = 330000
NTILE = 16
NCORE = 2
NW = NTILE * NCORE
EPT = 20736
NCH = EPT // 128
EP = EPT * NTILE
EPW = EP // NW
DEN_R = NP // 16


def _gat_dense_first(x, w, a_src, a_dst, dh):
    nb = NP // 512
    di = x.shape[1]
    do = 2 * dh

    kq = dh // 32

    def body(x_ref, w_ref, asr_ref, adr_ref, hs_ref, as_ref, ad_ref):
        h = jnp.dot(x_ref[...], w_ref[...], preferred_element_type=F32)
        for ci in range(2):
            for q in range(kq):
                lo = ci * dh + q * 32
                hs_ref[ci, q] = h[:, lo:lo + 32]
        as_ref[...] = jnp.dot(h, asr_ref[...], preferred_element_type=F32)
        ad_ref[...] = jnp.dot(h, adr_ref[...], preferred_element_type=F32)

    return pl.pallas_call(
        body,
        grid=(nb,),
        in_specs=[
            pl.BlockSpec((512, di), lambda i: (i, 0)),
            pl.BlockSpec((di, do), lambda i: (0, 0)),
            pl.BlockSpec((do, 1), lambda i: (0, 0)),
            pl.BlockSpec((do, 1), lambda i: (0, 0)),
        ],
        out_specs=[
            pl.BlockSpec((2, kq, 512, 32), lambda i: (0, 0, i, 0)),
            pl.BlockSpec((512, 1), lambda i: (i, 0)),
            pl.BlockSpec((512, 1), lambda i: (i, 0)),
        ],
        out_shape=[
            jax.ShapeDtypeStruct((2, kq, NP, 32), F32),
            jax.ShapeDtypeStruct((NP, 1), F32),
            jax.ShapeDtypeStruct((NP, 1), F32),
        ],
    )(x, w, a_src, a_dst)


def _gat_dense_mid(s_prev, b_prev, w, a_src, a_dst, dh):
    nb = NP // 512
    kp = s_prev.shape[1]
    dp = kp * 32
    di = 2 * dp
    kq = dh // 32

    def body(s_ref, b_ref, w_ref, asr_ref, adr_ref, hs_ref, as_ref, ad_ref):
        x0 = jnp.concatenate(
            [jnp.maximum(s_ref[0, q] + b_ref[0, q], 0.0) for q in range(kp)],
            axis=1)
        x1 = jnp.concatenate(
            [jnp.maximum(s_ref[1, q] + b_ref[1, q], 0.0) for q in range(kp)],
            axis=1)
        h = (jnp.dot(x0, w_ref[0:dp, :], preferred_element_type=F32)
             + jnp.dot(x1, w_ref[dp:di, :], preferred_element_type=F32))
        for ci in range(2):
            for q in range(kq):
                lo = ci * dh + q * 32
                hs_ref[ci, q] = h[:, lo:lo + 32]
        as_ref[...] = jnp.dot(h, asr_ref[...], preferred_element_type=F32)
        ad_ref[...] = jnp.dot(h, adr_ref[...], preferred_element_type=F32)

    return pl.pallas_call(
        body,
        grid=(nb,),
        in_specs=[
            pl.BlockSpec((2, kp, 512, 32), lambda i: (0, 0, i, 0)),
            pl.BlockSpec((2, kp, 1, 32), lambda i: (0, 0, 0, 0)),
            pl.BlockSpec((di, 2 * dh), lambda i: (0, 0)),
            pl.BlockSpec((2 * dh, 1), lambda i: (0, 0)),
            pl.BlockSpec((2 * dh, 1), lambda i: (0, 0)),
        ],
        out_specs=[
            pl.BlockSpec((2, kq, 512, 32), lambda i: (0, 0, i, 0)),
            pl.BlockSpec((512, 1), lambda i: (i, 0)),
            pl.BlockSpec((512, 1), lambda i: (i, 0)),
        ],
        out_shape=[
            jax.ShapeDtypeStruct((2, kq, NP, 32), F32),
            jax.ShapeDtypeStruct((NP, 1), F32),
            jax.ShapeDtypeStruct((NP, 1), F32),
        ],
    )(s_prev, b_prev, w, a_src, a_dst)


def _sc_pass_a(src_f, dst_f, as_f, ad_f, rid):
    mesh = plsc.VectorSubcoreMesh(core_axis_name="c", subcore_axis_name="s")

    @functools.partial(
        pl.kernel,
        mesh=mesh,
        compiler_params=pltpu.CompilerParams(needs_layout_passes=False, use_tc_tiling_on_sc=False),
        out_type=[
            jax.ShapeDtypeStruct((EP,), F32),
            jax.ShapeDtypeStruct((2, DEN_R, 16), F32),
        ],
        scratch_types=[
            pltpu.VMEM((EPW,), I32),
            pltpu.VMEM((EPW,), I32),
            pltpu.VMEM((EPW,), F32),
            pltpu.VMEM((NP,), F32),
            pltpu.VMEM((NP,), F32),
            pltpu.VMEM((DEN_R, 16), F32),
            pltpu.VMEM((5, 128), I32),
            pltpu.VMEM_SHARED((DEN_R, 16), F32),
        ],
    )
    def k(src_h, dst_h, as_h, ad_h, rid_h, ee_h, den_h,
          srcv, dstv, eev, asv, adv, denl, ridv, den_sh):
        c = lax.axis_index("c")
        s = lax.axis_index("s")
        w = s * NCORE + c
        off = w * EPW
        pltpu.sync_copy(as_h, asv)
        pltpu.sync_copy(ad_h, adv)
        pltpu.sync_copy(src_h.at[pl.ds(off, EPW)], srcv)
        pltpu.sync_copy(dst_h.at[pl.ds(off, EPW)], dstv)
        pltpu.sync_copy(rid_h, ridv)

        zero16 = jnp.zeros((16,), F32)

        def zloop(i, carry):
            denl[i] = zero16
            return carry

        lax.fori_loop(0, DEN_R, zloop, 0)

        def eloop(v, carry):
            sv = srcv[pl.ds(v * 16, 16)]
            dv = dstv[pl.ds(v * 16, 16)]
            a1 = plsc.load_gather(asv, [sv])
            a2 = plsc.load_gather(adv, [dv])
            e = a1 + a2
            e = jnp.maximum(e, e * 0.2)
            ee = jnp.exp(e)
            eev[pl.ds(v * 16, 16)] = ee
            plsc.addupdate_scatter(
                denl, [jnp.right_shift(dv, 4), jnp.bitwise_and(dv, 15)], ee)
            return carry

        lax.fori_loop(0, EPW // 16, eloop, 0)

        pltpu.sync_copy(eev, ee_h.at[pl.ds(off, EPW)])

        @pl.when(s == 0)
        def _():
            pltpu.sync_copy(denl, den_sh)

        plsc.subcore_barrier()

        @pl.when(s != 0)
        def _():
            for j in range(5):
                pltpu.sync_copy(denl.at[pl.ds(j * 128, 128)],
                                den_sh.at[ridv.at[j]], add=True)

        plsc.subcore_barrier()
        rows = DEN_R // NTILE
        pltpu.sync_copy(den_sh.at[pl.ds(s * rows, rows)],
                        den_h.at[c, pl.ds(s * rows, rows)])

    return k(src_f, dst_f, as_f, ad_f, rid)


def _sc_pass_b(src3, dst3, ee_f, den2, h_split, dh):
    mesh = plsc.VectorSubcoreMesh(core_axis_name="c", subcore_axis_name="s")

    kq = dh // 32

    @functools.partial(
        pl.kernel,
        mesh=mesh,
        compiler_params=pltpu.CompilerParams(needs_layout_passes=False,
                                             use_tc_tiling_on_sc=False),
        out_type=[
            jax.ShapeDtypeStruct((EP,), F32),
            jax.ShapeDtypeStruct((2, kq, NP, 32), F32),
        ],
        scratch_types=[
            pltpu.VMEM((NCH, 128), I32),
            pltpu.VMEM((NCH, 128), I32),
            pltpu.VMEM((EPT,), F32),
            pltpu.VMEM((DEN_R, 16), F32),
            pltpu.VMEM((DEN_R, 16), F32),
            pltpu.VMEM((128, 32), F32),
            pltpu.VMEM_SHARED((NR, 32), F32),
            pltpu.SemaphoreType.DMA,
        ],
    )
    def k(src_h, dst_h, ee_h, den_h, hs_h, alpha_h, s_h,
          srcv, dstv, av, d0v, d1v, rowb, out_sh, sem):
        c = lax.axis_index("c")
        s = lax.axis_index("s")
        off = s * EPT
        pltpu.sync_copy(src_h.at[s], srcv)
        pltpu.sync_copy(dst_h.at[s], dstv)
        pltpu.sync_copy(ee_h.at[pl.ds(off, EPT)], av)
        pltpu.sync_copy(den_h.at[0], d0v)
        pltpu.sync_copy(den_h.at[1], d1v)

        def aloop(j, carry):
            for t in range(8):
                dv = dstv[j, pl.ds(t * 16, 16)]
                hi = jnp.right_shift(dv, 4)
                lo = jnp.bitwise_and(dv, 15)
                d0 = plsc.load_gather(d0v, [hi, lo])
                d1 = plsc.load_gather(d1v, [hi, lo])
                ee = av[pl.ds(j * 128 + t * 16, 16)]
                av[pl.ds(j * 128 + t * 16, 16)] = ee / (d0 + d1)
            return carry

        lax.fori_loop(0, NCH, aloop, 0)

        @pl.when(c == 0)
        def _():
            pltpu.sync_copy(av, alpha_h.at[pl.ds(off, EPT)])

        zero16 = jnp.zeros((16,), F32)
        base = s * NRT

        for q in range(kq):
            def zloop(i, carry):
                rowb[i, pl.ds(0, 16)] = zero16
                rowb[i, pl.ds(16, 16)] = zero16
                return carry

            lax.fori_loop(0, 128, zloop, 0)
            for z in range(4):
                pltpu.sync_copy(rowb, out_sh.at[pl.ds(base + z * 128, 128)])
            pltpu.sync_copy(rowb.at[pl.ds(0, NRT - 512)],
                            out_sh.at[pl.ds(base + 512, NRT - 512)])
            plsc.subcore_barrier()

            def chunk(j, carry):
                pltpu.async_copy(hs_h.at[c, q].at[srcv.at[j]], rowb,
                                 sem).wait()

                def scale(rb, carry2):
                    a16 = av[pl.ds(j * 128 + rb * 16, 16)]
                    for u in range(16):
                        a = a16[u]
                        r = rb * 16 + u
                        rowb[r, pl.ds(0, 16)] = rowb[r, pl.ds(0, 16)] * a
                        rowb[r, pl.ds(16, 16)] = rowb[r, pl.ds(16, 16)] * a
                    return carry2

                lax.fori_loop(0, 8, scale, 0)
                pltpu.sync_copy(rowb, out_sh.at[dstv.at[j]], add=True)
                return carry

            lax.fori_loop(0, NCH, chunk, 0)

            plsc.subcore_barrier()
            pltpu.sync_copy(out_sh.at[pl.ds(base, NRT)],
                            s_h.at[c, q, pl.ds(base, NRT)])

    return k(src3, dst3, ee_f, den2, h_split)


def _lstm_u_first(s3, b3, wt, bias):
    nb = NP // 512
    kp = s3.shape[1]
    dp = kp * 32

    def body(s_ref, b_ref, w_ref, bias_ref, u_ref):
        x0 = jnp.concatenate(
            [jnp.maximum(s_ref[0, q] + b_ref[0, q], 0.0) for q in range(kp)],
            axis=1)
        x1 = jnp.concatenate(
            [jnp.maximum(s_ref[1, q] + b_ref[1, q], 0.0) for q in range(kp)],
            axis=1)
        u_ref[...] = (jnp.dot(x0, w_ref[0:dp, :], preferred_element_type=F32)
                      + jnp.dot(x1, w_ref[dp:2 * dp, :],
                                preferred_element_type=F32)
                      + bias_ref[...])

    return pl.pallas_call(
        body,
        grid=(nb,),
        in_specs=[
            pl.BlockSpec((2, kp, 512, 32), lambda i: (0, 0, i, 0)),
            pl.BlockSpec((2, kp, 1, 32), lambda i: (0, 0, 0, 0)),
            pl.BlockSpec((2 * dp, 1024), lambda i: (0, 0)),
            pl.BlockSpec((1, 1024), lambda i: (0, 0)),
        ],
        out_specs=pl.BlockSpec((512, 1024), lambda i: (i, 0)),
        out_shape=jax.ShapeDtypeStruct((NP, 1024), F32),
    )(s3, b3, wt, bias)


def _lstm_u_mid(yf, yb, wt, bias):
    nb = NP // 512

    def body(yf_ref, yb_ref, w_ref, bias_ref, u_ref):
        u_ref[...] = (jnp.dot(yf_ref[...], w_ref[0:128, :],
                              preferred_element_type=F32)
                      + jnp.dot(yb_ref[...], w_ref[128:256, :],
                                preferred_element_type=F32)
                      + bias_ref[...])

    return pl.pallas_call(
        body,
        grid=(nb,),
        in_specs=[
            pl.BlockSpec((512, 128), lambda i: (i, 0)),
            pl.BlockSpec((512, 128), lambda i: (i, 0)),
            pl.BlockSpec((256, 1024), lambda i: (0, 0)),
            pl.BlockSpec((1, 1024), lambda i: (0, 0)),
        ],
        out_specs=pl.BlockSpec((512, 1024), lambda i: (i, 0)),
        out_shape=jax.ShapeDtypeStruct((NP, 1024), F32),
    )(yf, yb, wt, bias)


def _lstm_rec(u, whf, whb):
    tb = 1000
    nblk = N // tb

    def body(uf_ref, ub_ref, wf_ref, wb_ref, yf_ref, yb_ref, st_s):
        i = pl.program_id(0)

        @pl.when(i == 0)
        def _():
            st_s[...] = jnp.zeros((8, 128), F32)

        def step(r, carry):
            hf, hb, cf, cb = carry
            gf = (jnp.dot(hf, wf_ref[...], preferred_element_type=F32)
                  + uf_ref[pl.ds(r, 1), :])
            gb = (jnp.dot(hb, wb_ref[...], preferred_element_type=F32)
                  + ub_ref[pl.ds(tb - 1 - r, 1), :])
            sf = jax.nn.sigmoid(gf[:, 0:384])
            sb = jax.nn.sigmoid(gb[:, 0:384])
            gtf = jnp.tanh(gf[:, 384:512])
            gtb = jnp.tanh(gb[:, 384:512])
            cf = sf[:, 128:256] * cf + sf[:, 0:128] * gtf
            cb = sb[:, 128:256] * cb + sb[:, 0:128] * gtb
            hf = sf[:, 256:384] * jnp.tanh(cf)
            hb = sb[:, 256:384] * jnp.tanh(cb)
            yf_ref[pl.ds(r, 1), :] = hf
            yb_ref[pl.ds(tb - 1 - r, 1), :] = hb
            return hf, hb, cf, cb

        init = (st_s[0:1, :], st_s[1:2, :], st_s[2:3, :], st_s[3:4, :])
        hf, hb, cf, cb = lax.fori_loop(0, tb, step, init, unroll=4)
        st_s[0:1, :] = hf
        st_s[1:2, :] = hb
        st_s[2:3, :] = cf
        st_s[3:4, :] = cb

    return pl.pallas_call(
        body,
        grid=(nblk,),
        in_specs=[
            pl.BlockSpec((tb, 512), lambda i: (i, 0)),
            pl.BlockSpec((tb, 512), lambda i: (nblk - 1 - i, 1)),
            pl.BlockSpec((128, 512), lambda i: (0, 0)),
            pl.BlockSpec((128, 512), lambda i: (0, 0)),
        ],
        out_specs=[
            pl.BlockSpec((tb, 128), lambda i: (i, 0)),
            pl.BlockSpec((tb, 128), lambda i: (nblk - 1 - i, 0)),
        ],
        out_shape=[
            jax.ShapeDtypeStruct((NP, 128), F32),
            jax.ShapeDtypeStruct((NP, 128), F32),
        ],
        scratch_shapes=[
            pltpu.VMEM((8, 128), F32),
        ],
    )(u, u, whf, whb)


def _pool_fc(yf, yb, bmat, fc1w, fc1b, g1, b1, fc2w, fc2b, g2, b2,
             fc3w, fc3b):
    rb = 400
    nb = N // rb
    bn_scale = float(1.0 / np.sqrt(1.0 + 1e-5))

    def body(yf_ref, yb_ref, b_ref, fc1w_ref, fc1b_ref, g1_ref, b1_ref,
             fc2w_ref, fc2b_ref, g2_ref, b2_ref, fc3w_ref, fc3b_ref,
             z_ref, acc):
        i = pl.program_id(0)

        @pl.when(i == 0)
        def _():
            acc[...] = jnp.zeros((32, 256), F32)

        dn = (((0,), (0,)), ((), ()))
        acc[:, 0:128] += lax.dot_general(b_ref[...], yf_ref[...], dn,
                                         preferred_element_type=F32)
        acc[:, 128:256] += lax.dot_general(b_ref[...], yb_ref[...], dn,
                                           preferred_element_type=F32)

        @pl.when(i == nb - 1)
        def _():
            p = acc[...] * (1.0 / 500.0)
            z = jnp.dot(p, fc1w_ref[...], preferred_element_type=F32)
            z = jnp.maximum(z + fc1b_ref[...], 0.0)
            z = z * bn_scale * g1_ref[...] + b1_ref[...]
            z = jnp.dot(z, fc2w_ref[...], preferred_element_type=F32)
            z = jnp.maximum(z + fc2b_ref[...], 0.0)
            z = z * bn_scale * g2_ref[...] + b2_ref[...]
            z = jnp.dot(z, fc3w_ref[...], preferred_element_type=F32)
            z_ref[...] = z[0:20, :] + fc3b_ref[...]

    return pl.pallas_call(
        body,
        grid=(nb,),
        in_specs=[
            pl.BlockSpec((rb, 128), lambda i: (i, 0)),
            pl.BlockSpec((rb, 128), lambda i: (i, 0)),
            pl.BlockSpec((rb, 32), lambda i: (i, 0)),
            pl.BlockSpec((256, 256), lambda i: (0, 0)),
            pl.BlockSpec((1, 256), lambda i: (0, 0)),
            pl.BlockSpec((1, 256), lambda i: (0, 0)),
            pl.BlockSpec((1, 256), lambda i: (0, 0)),
            pl.BlockSpec((256, 64), lambda i: (0, 0)),
            pl.BlockSpec((1, 64), lambda i: (0, 0)),
            pl.BlockSpec((1, 64), lambda i: (0, 0)),
            pl.BlockSpec((1, 64), lambda i: (0, 0)),
            pl.BlockSpec((64, 64), lambda i: (0, 0)),
            pl.BlockSpec((1, 64), lambda i: (0, 0)),
        ],
        out_specs=pl.BlockSpec((20, 64), lambda i: (0, 0)),
        out_shape=jax.ShapeDtypeStruct((20, 64), F32),
        scratch_shapes=[pltpu.VMEM((32, 256), F32)],
    )(yf, yb, bmat, fc1w, fc1b, g1, b1, fc2w, fc2b, g2, b2, fc3w, fc3b)


def _gat_layer(x_or_s, b_prev, params, idx, layer, dh, first):
    w = params['W%d' % layer]
    a_src = params['a_src%d' % layer].reshape(-1, 1)
    a_dst = params['a_dst%d' % layer].reshape(-1, 1)
    if first:
        hs, as_, ad_ = _gat_dense_first(x_or_s, w, a_src, a_dst, dh)
    else:
        hs, as_, ad_ = _gat_dense_mid(x_or_s, b_prev, w, a_src, a_dst, dh)
    src_f, dst_f, src3, dst3, rid = idx
    ee, den2 = _sc_pass_a(src_f, dst_f, as_.reshape(NP), ad_.reshape(NP),
                          rid)
    alpha, s_out = _sc_pass_b(src3, dst3, ee, den2, hs, dh)
    return alpha, s_out


def kernel(x, edge_index, params):
    loop = jnp.arange(N, dtype=edge_index.dtype)
    ei = jnp.concatenate([edge_index, jnp.stack([loop, loop])], axis=1)

    src_f = jnp.concatenate([ei[0], jnp.zeros((EP - E,), I32)])
    dst_f = jnp.concatenate([ei[1], jnp.full((EP - E,), N, I32)])
    src3 = src_f.reshape(NTILE, NCH, 128)
    dst3 = dst_f.reshape(NTILE, NCH, 128)
    rid = jnp.arange(DEN_R, dtype=I32).reshape(5, 128)
    idx = (src_f, dst_f, src3, dst3, rid)

    x_p = jnp.pad(x, ((0, NP - N), (0, 0)))

    a1, s1 = _gat_layer(x_p, None, params, idx, 1, 64, True)
    b1 = params['b1'].reshape(2, 2, 1, 32)
    a2, s2 = _gat_layer(s1, b1, params, idx, 2, 128, False)
    b2 = params['b2'].reshape(2, 4, 1, 32)
    a3, s3 = _gat_layer(s2, b2, params, idx, 3, 64, False)
    b3 = params['b3'].reshape(2, 2, 1, 32)

    def _gp(m):
        return jnp.concatenate([m[0:256], m[384:512], m[256:384]], axis=0)

    w0 = jnp.concatenate([_gp(params['Wih_l0f']),
                          _gp(params['Wih_l0b'])], axis=0).T
    bias0 = (jnp.concatenate([_gp(params['bih_l0f'] + params['bhh_l0f']),
                              _gp(params['bih_l0b'] + params['bhh_l0b'])])
             .reshape(1, 1024))
    u0 = _lstm_u_first(s3, b3, w0, bias0)
    y0f, y0b = _lstm_rec(u0, _gp(params['Whh_l0f']).T,
                         _gp(params['Whh_l0b']).T)

    w1 = jnp.concatenate([_gp(params['Wih_l1f']),
                          _gp(params['Wih_l1b'])], axis=0).T
    bias1 = (jnp.concatenate([_gp(params['bih_l1f'] + params['bhh_l1f']),
                              _gp(params['bih_l1b'] + params['bhh_l1b'])])
             .reshape(1, 1024))
    u1 = _lstm_u_mid(y0f, y0b, w1, bias1)
    y1f, y1b = _lstm_rec(u1, _gp(params['Whh_l1f']).T,
                         _gp(params['Whh_l1b']).T)

    row = jnp.arange(NP, dtype=I32)
    grp = row // 500
    bmat = jnp.where((row[:, None] < N) & (grp[:, None] == jnp.arange(32)),
                     1.0, 0.0).astype(F32)
    z = _pool_fc(y1f, y1b, bmat,
                 params['fc1_w'], params['fc1_b'].reshape(1, 256),
                 params['bn1_g'].reshape(1, 256),
                 params['bn1_b'].reshape(1, 256),
                 params['fc2_w'], params['fc2_b'].reshape(1, 64),
                 params['bn2_g'].reshape(1, 64),
                 params['bn2_b'].reshape(1, 64),
                 params['fc3_w'], params['fc3_b'].reshape(1, 64))

    return (z, a1[:E], a2[:E], a3[:E], ei)
